# baseline (speedup 1.0000x reference)
import functools

import jax
import jax.numpy as jnp
from jax.experimental import pallas as pl

_EPS = 1e-12


def _embed_ln_kernel(ids_ref, tok_aug_ref, tokT_ref, pos_ref, gamma_ref,
                     out_ref, *, vocab: int, kpad: int):
    b, sblk, _ = ids_ref.shape
    d = tokT_ref.shape[0]
    inv_d = 1.0 / d
    tok_aug = tok_aug_ref[...]
    tok_t = tokT_ref[...]
    pos = pos_ref[...]
    g = gamma_ref[...]

    pos_mean = jnp.mean(pos, axis=1, keepdims=True)
    pos_sq = jnp.mean(pos * pos, axis=1, keepdims=True)
    pos_g = pos * g
    tok_mean = jnp.mean(tok_t, axis=0, keepdims=True)
    tok_sq = jnp.mean(tok_t * tok_t, axis=0, keepdims=True)
    cross = jnp.dot(pos, tok_t, preferred_element_type=jnp.float32) * inv_d
    sv = tok_sq + 2.0 * cross

    iota32 = jax.lax.broadcasted_iota(jnp.int32, (sblk, vocab), 1)
    iotak = jax.lax.broadcasted_iota(jnp.int32, (sblk, kpad), 1)
    for bi in range(b):
        ids = ids_ref[bi]
        onehot = (ids == iota32).astype(jnp.float32)
        m = pos_mean + jnp.sum(onehot * tok_mean, axis=1, keepdims=True)
        ex2 = pos_sq + jnp.sum(onehot * sv, axis=1, keepdims=True)
        rstd = jax.lax.rsqrt(ex2 - m * m + _EPS)
        lhs = jnp.where(iotak == ids, rstd, 0.0)
        lhs = jnp.where(iotak == vocab, -m * rstd, lhs)
        lhs = jnp.where(iotak == vocab + 1, 1.0, lhs)
        mm = jnp.dot(lhs, tok_aug, preferred_element_type=jnp.float32)
        out_ref[bi] = mm + pos_g * rstd


def kernel(input_ids, tok_emb, pos_emb, gamma, beta):
    b, s = input_ids.shape
    vocab, d = tok_emb.shape
    sblk = 1024
    kpad = 40
    grid = (s // sblk,)

    ids = input_ids.astype(jnp.int32).reshape(b, s, 1)
    pos = pos_emb[:s]
    tok_t = tok_emb.T
    tok_aug = jnp.zeros((kpad, d), jnp.float32)
    tok_aug = tok_aug.at[:vocab].set(tok_emb * gamma[None, :])
    tok_aug = tok_aug.at[vocab].set(gamma)
    tok_aug = tok_aug.at[vocab + 1].set(beta)

    out = pl.pallas_call(
        functools.partial(_embed_ln_kernel, vocab=vocab, kpad=kpad),
        grid=grid,
        in_specs=[
            pl.BlockSpec((b, sblk, 1), lambda i: (0, i, 0)),
            pl.BlockSpec((kpad, d), lambda i: (0, 0)),
            pl.BlockSpec((d, vocab), lambda i: (0, 0)),
            pl.BlockSpec((sblk, d), lambda i: (i, 0)),
            pl.BlockSpec((d,), lambda i: (0,)),
        ],
        out_specs=pl.BlockSpec((b, sblk, d), lambda i: (0, i, 0)),
        out_shape=jax.ShapeDtypeStruct((b, s, d), jnp.float32),
    )(ids, tok_aug, tok_t, pos, gamma)
    return out


# speedup vs baseline: 1.1400x; 1.1400x over previous
import functools

import jax
import jax.numpy as jnp
from jax.experimental import pallas as pl

_EPS = 1e-12


def _embed_ln_kernel(ids_ref, tok_aug_ref, tokT_ref, pos_ref, gamma_ref,
                     beta_ref, out_ref, *, vocab: int, kpad: int):
    b, sblk, _ = ids_ref.shape
    d = tokT_ref.shape[0]
    inv_d = 1.0 / d
    tok_aug = tok_aug_ref[...]
    tok_t = tokT_ref[...]
    pos = pos_ref[...]
    g = gamma_ref[...]

    pos_mean = jnp.mean(pos, axis=1, keepdims=True)
    pos_sq = jnp.mean(pos * pos, axis=1, keepdims=True)
    pos_g = pos * g
    tok_mean = jnp.mean(tok_t, axis=0, keepdims=True)
    tok_sq = jnp.mean(tok_t * tok_t, axis=0, keepdims=True)
    cross = jnp.dot(pos, tok_t, preferred_element_type=jnp.float32) * inv_d
    sv = tok_sq + 2.0 * cross

    beta = beta_ref[...]
    iota32 = jax.lax.broadcasted_iota(jnp.int32, (sblk, vocab), 1)
    iotak = jax.lax.broadcasted_iota(jnp.int32, (sblk, kpad), 1)
    for bi in range(b):
        ids = ids_ref[bi]
        onehot = (ids == iota32).astype(jnp.float32)
        m = pos_mean + jnp.sum(onehot * tok_mean, axis=1, keepdims=True)
        ex2 = pos_sq + jnp.sum(onehot * sv, axis=1, keepdims=True)
        rstd = jax.lax.rsqrt(ex2 - m * m + _EPS)
        lhs = jnp.where(iotak == ids, 1.0, 0.0)
        lhs = jnp.where(iotak == vocab, -m, lhs)
        mm = jnp.dot(lhs, tok_aug, preferred_element_type=jnp.float32)
        out_ref[bi] = (mm + pos_g) * rstd + beta


def kernel(input_ids, tok_emb, pos_emb, gamma, beta):
    b, s = input_ids.shape
    vocab, d = tok_emb.shape
    sblk = 1024
    kpad = 40
    grid = (s // sblk,)

    ids = input_ids.astype(jnp.int32).reshape(b, s, 1)
    pos = pos_emb[:s]
    tok_t = tok_emb.T
    tok_aug = jnp.zeros((kpad, d), jnp.float32)
    tok_aug = tok_aug.at[:vocab].set(tok_emb * gamma[None, :])
    tok_aug = tok_aug.at[vocab].set(gamma)

    out = pl.pallas_call(
        functools.partial(_embed_ln_kernel, vocab=vocab, kpad=kpad),
        grid=grid,
        in_specs=[
            pl.BlockSpec((b, sblk, 1), lambda i: (0, i, 0)),
            pl.BlockSpec((kpad, d), lambda i: (0, 0)),
            pl.BlockSpec((d, vocab), lambda i: (0, 0)),
            pl.BlockSpec((sblk, d), lambda i: (i, 0)),
            pl.BlockSpec((d,), lambda i: (0,)),
            pl.BlockSpec((d,), lambda i: (0,)),
        ],
        out_specs=pl.BlockSpec((b, sblk, d), lambda i: (0, i, 0)),
        out_shape=jax.ShapeDtypeStruct((b, s, d), jnp.float32),
    )(ids, tok_aug, tok_t, pos, gamma, beta)
    return out
